# baseline (speedup 1.0000x reference)
import jax
import jax.numpy as jnp
from jax import lax
from jax.experimental import pallas as pl
from jax.experimental.pallas import tpu as pltpu
from jax.experimental.pallas import tpu_sc as plsc

H = 56
W = 56
HW = H * W
NBLK = HW // 16
R = 64 * 256
HALF = 3

RB = 512
N_SC = 3072
N_TC = R - N_SC
SC_BLKS = N_SC // RB

NW = 32
ROWS_PER_W = N_SC // NW
CHUNK = 8
NCHUNK = ROWS_PER_W // CHUNK
NHALF = NCHUNK // 2


def _sc_body(x_hbm, t_hbm, out_hbm, tbuf, buf_a, buf_b, redf, redi,
             semi_a, semi_b, semo_a, semo_b):
    cid = lax.axis_index("c")
    sid = lax.axis_index("s")
    wid = sid * 2 + cid
    base = wid * ROWS_PER_W

    lane = lax.iota(jnp.int32, 16)
    zerov = jnp.zeros((16,), jnp.float32)

    def start_in(ci, buf, sem):
        ebase = (base + ci * CHUNK) * HW
        pltpu.async_copy(x_hbm.at[pl.ds(ebase, CHUNK * HW)],
                         buf.at[pl.ds(0, CHUNK * HW)], sem)

    def wait_in(buf, sem):
        pltpu.make_async_copy(x_hbm.at[pl.ds(0, CHUNK * HW)],
                              buf.at[pl.ds(0, CHUNK * HW)], sem).wait()

    def start_out(ci, buf, sem):
        ebase = (base + ci * CHUNK) * HW
        pltpu.async_copy(buf.at[pl.ds(0, CHUNK * HW)],
                         out_hbm.at[pl.ds(ebase, CHUNK * HW)], sem)

    def wait_out(buf, sem):
        pltpu.make_async_copy(buf.at[pl.ds(0, CHUNK * HW)],
                              out_hbm.at[pl.ds(0, CHUNK * HW)], sem).wait()

    start_in(0, buf_a, semi_a)
    start_in(1, buf_b, semi_b)
    pltpu.sync_copy(t_hbm.at[pl.ds(base, ROWS_PER_W)],
                    tbuf.at[pl.ds(0, ROWS_PER_W)])

    redf[pl.ds(16, 16)] = jnp.full((16,), -jnp.inf, jnp.float32)
    redi[pl.ds(16, 16)] = jnp.full((16,), HW, jnp.int32)

    def process_row(buf, roff):
        def amax_body(j, carry):
            m, bj = carry
            v = buf[pl.ds(roff + j * 16, 16)]
            gt = v > m
            return jnp.where(gt, v, m), jnp.where(gt, j, bj)

        m0 = jnp.full((16,), -jnp.inf, jnp.float32)
        b0 = jnp.zeros((16,), jnp.int32)
        m, bj = lax.fori_loop(0, NBLK, amax_body, (m0, b0), unroll=14)

        redf[pl.ds(0, 16)] = m
        redi[pl.ds(0, 16)] = bj * 16 + lane
        for sh in (8, 4, 2, 1):
            am = redf[pl.ds(0, 16)]
            ai = redi[pl.ds(0, 16)]
            bm = redf[pl.ds(sh, 16)]
            bi = redi[pl.ds(sh, 16)]
            better = (bm > am) | ((bm == am) & (bi < ai))
            redf[pl.ds(0, 16)] = jnp.where(better, bm, am)
            redi[pl.ds(0, 16)] = jnp.where(better, bi, ai)
        idx = redi[pl.ds(0, 16)][0]

        mh = idx // W
        mw = idx - mh * W
        h1 = jnp.maximum(mh - HALF, 0)
        h2 = jnp.minimum(mh + HALF, H - 1)
        w1 = jnp.maximum(mw - HALF, 0)
        w2 = jnp.minimum(mw + HALF, W - 1)
        area = (h2 - h1 + 1) * (w2 - w1 + 1)
        area_v = jnp.full((16,), 1.0, jnp.float32) * area.astype(jnp.float32)
        lamv = jnp.float32(HW) / (jnp.float32(HW) - area_v)

        def scale_body(j, _):
            sl = pl.ds(roff + j * 16, 16)
            buf[sl] = buf[sl] * lamv
            return 0

        lax.fori_loop(0, NBLK, scale_body, 0, unroll=14)

        msk = lane <= (w2 - w1)

        def zero_body(hr, _):
            sl = pl.ds(roff + hr * W + w1, 16)
            buf[sl] = jnp.where(msk, zerov, buf[sl])
            return 0

        lax.fori_loop(h1, h2 + 1, zero_body, 0)

    def compute(buf, ci):
        tv = tbuf[pl.ds(ci * CHUNK, 16)]
        for r in range(CHUNK):
            t = tv[r]

            @pl.when(t != 0.0)
            def _(roff=r * HW):
                process_row(buf, roff)

    def pair_body(i, _):
        g0 = 2 * i
        g1 = g0 + 1

        wait_in(buf_a, semi_a)
        compute(buf_a, g0)
        start_out(g0, buf_a, semo_a)

        wait_in(buf_b, semi_b)
        compute(buf_b, g1)
        start_out(g1, buf_b, semo_b)

        @pl.when(i < NHALF - 1)
        def _():
            wait_out(buf_a, semo_a)
            start_in(g0 + 2, buf_a, semi_a)
            wait_out(buf_b, semo_b)
            start_in(g1 + 2, buf_b, semi_b)

        return 0

    lax.fori_loop(0, NHALF, pair_body, 0)
    wait_out(buf_a, semo_a)
    wait_out(buf_b, semo_b)


def _sc_call(x1, t1):
    mesh = plsc.VectorSubcoreMesh(core_axis_name="c", subcore_axis_name="s")
    return pl.kernel(
        _sc_body,
        out_type=jax.ShapeDtypeStruct((N_SC * HW,), jnp.float32),
        mesh=mesh,
        scratch_types=[
            pltpu.VMEM((ROWS_PER_W + 16,), jnp.float32),
            pltpu.VMEM((CHUNK * HW + 16,), jnp.float32),
            pltpu.VMEM((CHUNK * HW + 16,), jnp.float32),
            pltpu.VMEM((32,), jnp.float32),
            pltpu.VMEM((32,), jnp.int32),
            pltpu.SemaphoreType.DMA,
            pltpu.SemaphoreType.DMA,
            pltpu.SemaphoreType.DMA,
            pltpu.SemaphoreType.DMA,
        ],
    )(x1, t1)


def _tc_body(x_ref, t_ref, o_ref):
    xb = x_ref[...]
    t = t_ref[...]
    m = jnp.max(xb, axis=1, keepdims=True)
    ii = lax.broadcasted_iota(jnp.int32, (RB, HW), 1)
    idx = jnp.min(jnp.where(xb == m, ii, HW), axis=1, keepdims=True)
    mh = idx // W
    mw = idx - mh * W
    h1 = jnp.maximum(mh - HALF, 0)
    h2 = jnp.minimum(mh + HALF, H - 1)
    w1 = jnp.maximum(mw - HALF, 0)
    w2 = jnp.minimum(mw + HALF, W - 1)
    area = (h2 - h1 + 1) * (w2 - w1 + 1)
    lam = jnp.float32(HW) / (jnp.float32(HW) - area.astype(jnp.float32))
    ch = ii // W
    cw = ii - ch * W
    inside = (ch >= h1) & (ch <= h2) & (cw >= w1) & (cw <= w2)
    res = jnp.where(inside, jnp.float32(0), xb * lam)
    o_ref[...] = jnp.where(t != 0.0, res, xb)


def _tc_call(x2, t2):
    return pl.pallas_call(
        _tc_body,
        grid=(N_TC // RB,),
        in_specs=[
            pl.BlockSpec((RB, HW), lambda i: (i + SC_BLKS, 0)),
            pl.BlockSpec((RB, 1), lambda i: (i + SC_BLKS, 0)),
        ],
        out_specs=pl.BlockSpec((RB, HW), lambda i: (i + SC_BLKS, 0)),
        out_shape=jax.ShapeDtypeStruct((R, HW), jnp.float32),
    )(x2, t2)


def kernel(x, T):
    b, c, h, w = x.shape
    x1 = x.reshape(R * HW)
    t1 = T.reshape(R)
    out_sc = _sc_call(x1, t1)
    out_tc = _tc_call(x.reshape(R, HW), T.reshape(R, 1))
    out = lax.dynamic_update_slice(out_tc, out_sc.reshape(N_SC, HW), (0, 0))
    return out.reshape(b, c, h, w)


# speedup vs baseline: 1.1958x; 1.1958x over previous
import jax
import jax.numpy as jnp
from jax import lax
from jax.experimental import pallas as pl
from jax.experimental.pallas import tpu as pltpu
from jax.experimental.pallas import tpu_sc as plsc

H = 56
W = 56
HW = H * W
NBLK = HW // 16
R = 64 * 256
HALF = 3

RB = 512
N_SC = 3072
N_TC = R - N_SC
SC_BLKS = N_SC // RB

NW = 32
ROWS_PER_W = N_SC // NW
CHUNK = 8
NCHUNK = ROWS_PER_W // CHUNK
NHALF = NCHUNK // 2


def _sc_body(x_hbm, t_hbm, out_hbm, tbuf, buf_a, buf_b, redf, redi,
             semi_a, semi_b, semo_a, semo_b):
    cid = lax.axis_index("c")
    sid = lax.axis_index("s")
    wid = sid * 2 + cid
    base = wid * ROWS_PER_W

    lane = lax.iota(jnp.int32, 16)
    zerov = jnp.zeros((16,), jnp.float32)

    def start_in(ci, buf, sem):
        ebase = (base + ci * CHUNK) * HW
        pltpu.async_copy(x_hbm.at[pl.ds(ebase, CHUNK * HW)],
                         buf.at[pl.ds(0, CHUNK * HW)], sem)

    def wait_in(buf, sem):
        pltpu.make_async_copy(x_hbm.at[pl.ds(0, CHUNK * HW)],
                              buf.at[pl.ds(0, CHUNK * HW)], sem).wait()

    def start_out(ci, buf, sem):
        ebase = (base + ci * CHUNK) * HW
        pltpu.async_copy(buf.at[pl.ds(0, CHUNK * HW)],
                         out_hbm.at[pl.ds(ebase, CHUNK * HW)], sem)

    def wait_out(buf, sem):
        pltpu.make_async_copy(buf.at[pl.ds(0, CHUNK * HW)],
                              out_hbm.at[pl.ds(0, CHUNK * HW)], sem).wait()

    start_in(0, buf_a, semi_a)
    start_in(1, buf_b, semi_b)
    pltpu.sync_copy(t_hbm.at[pl.ds(base, ROWS_PER_W)],
                    tbuf.at[pl.ds(0, ROWS_PER_W)])

    redf[pl.ds(16, 16)] = jnp.full((16,), -jnp.inf, jnp.float32)
    redi[pl.ds(16, 16)] = jnp.full((16,), HW, jnp.int32)

    def process_row(buf, roff):
        def amax_body(j, carry):
            m, bj = carry
            v = buf[pl.ds(roff + j * 16, 16)]
            gt = v > m
            return jnp.where(gt, v, m), jnp.where(gt, j, bj)

        m0 = jnp.full((16,), -jnp.inf, jnp.float32)
        b0 = jnp.zeros((16,), jnp.int32)
        m, bj = lax.fori_loop(0, NBLK, amax_body, (m0, b0), unroll=14)

        redf[pl.ds(0, 16)] = m
        redi[pl.ds(0, 16)] = bj * 16 + lane
        for sh in (8, 4, 2, 1):
            am = redf[pl.ds(0, 16)]
            ai = redi[pl.ds(0, 16)]
            bm = redf[pl.ds(sh, 16)]
            bi = redi[pl.ds(sh, 16)]
            better = (bm > am) | ((bm == am) & (bi < ai))
            redf[pl.ds(0, 16)] = jnp.where(better, bm, am)
            redi[pl.ds(0, 16)] = jnp.where(better, bi, ai)
        idx = redi[pl.ds(0, 16)][0]

        mh = idx // W
        mw = idx - mh * W
        h1 = jnp.maximum(mh - HALF, 0)
        h2 = jnp.minimum(mh + HALF, H - 1)
        w1 = jnp.maximum(mw - HALF, 0)
        w2 = jnp.minimum(mw + HALF, W - 1)
        area = (h2 - h1 + 1) * (w2 - w1 + 1)
        area_v = jnp.full((16,), 1.0, jnp.float32) * area.astype(jnp.float32)
        lamv = jnp.float32(HW) / (jnp.float32(HW) - area_v)

        def scale_body(j, _):
            sl = pl.ds(roff + j * 16, 16)
            buf[sl] = buf[sl] * lamv
            return 0

        lax.fori_loop(0, NBLK, scale_body, 0, unroll=14)

        msk = lane <= (w2 - w1)

        def zero_body(hr, _):
            sl = pl.ds(roff + hr * W + w1, 16)
            buf[sl] = jnp.where(msk, zerov, buf[sl])
            return 0

        lax.fori_loop(h1, h2 + 1, zero_body, 0)

    def compute(buf, ci):
        tv = tbuf[pl.ds(ci * CHUNK, 16)]
        for r in range(CHUNK):
            t = tv[r]

            @pl.when(t != 0.0)
            def _(roff=r * HW):
                process_row(buf, roff)

    def pair_body(i, _):
        g0 = 2 * i
        g1 = g0 + 1

        wait_in(buf_a, semi_a)
        compute(buf_a, g0)
        start_out(g0, buf_a, semo_a)

        wait_in(buf_b, semi_b)
        compute(buf_b, g1)
        start_out(g1, buf_b, semo_b)

        @pl.when(i < NHALF - 1)
        def _():
            wait_out(buf_a, semo_a)
            start_in(g0 + 2, buf_a, semi_a)
            wait_out(buf_b, semo_b)
            start_in(g1 + 2, buf_b, semi_b)

        return 0

    lax.fori_loop(0, NHALF, pair_body, 0)
    wait_out(buf_a, semo_a)
    wait_out(buf_b, semo_b)


def _sc_call(x1, t1):
    mesh = plsc.VectorSubcoreMesh(core_axis_name="c", subcore_axis_name="s")
    return pl.kernel(
        _sc_body,
        out_type=jax.ShapeDtypeStruct((N_SC * HW,), jnp.float32),
        mesh=mesh,
        scratch_types=[
            pltpu.VMEM((ROWS_PER_W + 16,), jnp.float32),
            pltpu.VMEM((CHUNK * HW + 16,), jnp.float32),
            pltpu.VMEM((CHUNK * HW + 16,), jnp.float32),
            pltpu.VMEM((32,), jnp.float32),
            pltpu.VMEM((32,), jnp.int32),
            pltpu.SemaphoreType.DMA,
            pltpu.SemaphoreType.DMA,
            pltpu.SemaphoreType.DMA,
            pltpu.SemaphoreType.DMA,
        ],
    )(x1, t1)


def _tc_body(x_ref, t_ref, eh_ref, ew_ref, ii_ref, o_ref):
    xb = x_ref[...]
    t = t_ref[...]
    ii = ii_ref[...]
    m = jnp.max(xb, axis=1, keepdims=True)
    idx = jnp.min(jnp.where(xb == m, ii, HW), axis=1, keepdims=True)
    mh = idx // W
    mw = idx - mh * W
    h1 = jnp.maximum(mh - HALF, 0)
    h2 = jnp.minimum(mh + HALF, H - 1)
    w1 = jnp.maximum(mw - HALF, 0)
    w2 = jnp.minimum(mw + HALF, W - 1)
    area = (h2 - h1 + 1) * (w2 - w1 + 1)
    lam = jnp.float32(HW) / (jnp.float32(HW) - area.astype(jnp.float32))
    on = t != 0.0
    lam_eff = jnp.where(on, lam, jnp.float32(1))
    a = jnp.where(on, lam, jnp.float32(0))
    hh = lax.broadcasted_iota(jnp.int32, (RB, H), 1)
    ww = lax.broadcasted_iota(jnp.int32, (RB, W), 1)
    rowm = ((hh >= h1) & (hh <= h2)).astype(jnp.float32) * a
    colm = ((ww >= w1) & (ww <= w2)).astype(jnp.float32)
    rowe = jnp.dot(rowm, eh_ref[...], preferred_element_type=jnp.float32,
                   precision=lax.Precision.HIGHEST)
    cole = jnp.dot(colm, ew_ref[...], preferred_element_type=jnp.float32,
                   precision=lax.Precision.HIGHEST)
    g = lam_eff - rowe * cole
    o_ref[...] = xb * g


def _expand_tables():
    fi = jnp.arange(HW, dtype=jnp.int32)
    eh = (fi[None, :] // W == jnp.arange(H, dtype=jnp.int32)[:, None])
    ew = (fi[None, :] % W == jnp.arange(W, dtype=jnp.int32)[:, None])
    return (eh.astype(jnp.float32), ew.astype(jnp.float32),
            fi[None, :])


def _tc_call(x2, t2):
    eh, ew, ii = _expand_tables()
    return pl.pallas_call(
        _tc_body,
        grid=(R // RB,),
        in_specs=[
            pl.BlockSpec((RB, HW), lambda i: (i, 0)),
            pl.BlockSpec((RB, 1), lambda i: (i, 0)),
            pl.BlockSpec((H, HW), lambda i: (0, 0)),
            pl.BlockSpec((W, HW), lambda i: (0, 0)),
            pl.BlockSpec((1, HW), lambda i: (0, 0)),
        ],
        out_specs=pl.BlockSpec((RB, HW), lambda i: (i, 0)),
        out_shape=jax.ShapeDtypeStruct((R, HW), jnp.float32),
    )(x2, t2, eh, ew, ii)


def kernel(x, T):
    b, c, h, w = x.shape
    out = _tc_call(x.reshape(R, HW), T.reshape(R, 1))
    return out.reshape(b, c, h, w)


# speedup vs baseline: 2.1821x; 1.8248x over previous
import jax
import jax.numpy as jnp
from jax import lax
from jax.experimental import pallas as pl
from jax.experimental.pallas import tpu as pltpu
from jax.experimental.pallas import tpu_sc as plsc

H = 56
W = 56
HW = H * W
NBLK = HW // 16
R = 64 * 256
HALF = 3

RB = 128
N_SC = 3072
N_TC = R - N_SC
SC_BLKS = N_SC // RB

NW = 32
ROWS_PER_W = N_SC // NW
CHUNK = 8
NCHUNK = ROWS_PER_W // CHUNK
NHALF = NCHUNK // 2


def _sc_body(x_hbm, t_hbm, out_hbm, tbuf, buf_a, buf_b, redf, redi,
             semi_a, semi_b, semo_a, semo_b):
    cid = lax.axis_index("c")
    sid = lax.axis_index("s")
    wid = sid * 2 + cid
    base = wid * ROWS_PER_W

    lane = lax.iota(jnp.int32, 16)
    zerov = jnp.zeros((16,), jnp.float32)

    def start_in(ci, buf, sem):
        ebase = (base + ci * CHUNK) * HW
        pltpu.async_copy(x_hbm.at[pl.ds(ebase, CHUNK * HW)],
                         buf.at[pl.ds(0, CHUNK * HW)], sem)

    def wait_in(buf, sem):
        pltpu.make_async_copy(x_hbm.at[pl.ds(0, CHUNK * HW)],
                              buf.at[pl.ds(0, CHUNK * HW)], sem).wait()

    def start_out(ci, buf, sem):
        ebase = (base + ci * CHUNK) * HW
        pltpu.async_copy(buf.at[pl.ds(0, CHUNK * HW)],
                         out_hbm.at[pl.ds(ebase, CHUNK * HW)], sem)

    def wait_out(buf, sem):
        pltpu.make_async_copy(buf.at[pl.ds(0, CHUNK * HW)],
                              out_hbm.at[pl.ds(0, CHUNK * HW)], sem).wait()

    start_in(0, buf_a, semi_a)
    start_in(1, buf_b, semi_b)
    pltpu.sync_copy(t_hbm.at[pl.ds(base, ROWS_PER_W)],
                    tbuf.at[pl.ds(0, ROWS_PER_W)])

    redf[pl.ds(16, 16)] = jnp.full((16,), -jnp.inf, jnp.float32)
    redi[pl.ds(16, 16)] = jnp.full((16,), HW, jnp.int32)

    def process_row(buf, roff):
        def amax_body(j, carry):
            m, bj = carry
            v = buf[pl.ds(roff + j * 16, 16)]
            gt = v > m
            return jnp.where(gt, v, m), jnp.where(gt, j, bj)

        m0 = jnp.full((16,), -jnp.inf, jnp.float32)
        b0 = jnp.zeros((16,), jnp.int32)
        m, bj = lax.fori_loop(0, NBLK, amax_body, (m0, b0), unroll=14)

        redf[pl.ds(0, 16)] = m
        redi[pl.ds(0, 16)] = bj * 16 + lane
        for sh in (8, 4, 2, 1):
            am = redf[pl.ds(0, 16)]
            ai = redi[pl.ds(0, 16)]
            bm = redf[pl.ds(sh, 16)]
            bi = redi[pl.ds(sh, 16)]
            better = (bm > am) | ((bm == am) & (bi < ai))
            redf[pl.ds(0, 16)] = jnp.where(better, bm, am)
            redi[pl.ds(0, 16)] = jnp.where(better, bi, ai)
        idx = redi[pl.ds(0, 16)][0]

        mh = idx // W
        mw = idx - mh * W
        h1 = jnp.maximum(mh - HALF, 0)
        h2 = jnp.minimum(mh + HALF, H - 1)
        w1 = jnp.maximum(mw - HALF, 0)
        w2 = jnp.minimum(mw + HALF, W - 1)
        area = (h2 - h1 + 1) * (w2 - w1 + 1)
        area_v = jnp.full((16,), 1.0, jnp.float32) * area.astype(jnp.float32)
        lamv = jnp.float32(HW) / (jnp.float32(HW) - area_v)

        def scale_body(j, _):
            sl = pl.ds(roff + j * 16, 16)
            buf[sl] = buf[sl] * lamv
            return 0

        lax.fori_loop(0, NBLK, scale_body, 0, unroll=14)

        msk = lane <= (w2 - w1)

        def zero_body(hr, _):
            sl = pl.ds(roff + hr * W + w1, 16)
            buf[sl] = jnp.where(msk, zerov, buf[sl])
            return 0

        lax.fori_loop(h1, h2 + 1, zero_body, 0)

    def compute(buf, ci):
        tv = tbuf[pl.ds(ci * CHUNK, 16)]
        for r in range(CHUNK):
            t = tv[r]

            @pl.when(t != 0.0)
            def _(roff=r * HW):
                process_row(buf, roff)

    def pair_body(i, _):
        g0 = 2 * i
        g1 = g0 + 1

        wait_in(buf_a, semi_a)
        compute(buf_a, g0)
        start_out(g0, buf_a, semo_a)

        wait_in(buf_b, semi_b)
        compute(buf_b, g1)
        start_out(g1, buf_b, semo_b)

        @pl.when(i < NHALF - 1)
        def _():
            wait_out(buf_a, semo_a)
            start_in(g0 + 2, buf_a, semi_a)
            wait_out(buf_b, semo_b)
            start_in(g1 + 2, buf_b, semi_b)

        return 0

    lax.fori_loop(0, NHALF, pair_body, 0)
    wait_out(buf_a, semo_a)
    wait_out(buf_b, semo_b)


def _sc_call(x1, t1):
    mesh = plsc.VectorSubcoreMesh(core_axis_name="c", subcore_axis_name="s")
    return pl.kernel(
        _sc_body,
        out_type=jax.ShapeDtypeStruct((N_SC * HW,), jnp.float32),
        mesh=mesh,
        scratch_types=[
            pltpu.VMEM((ROWS_PER_W + 16,), jnp.float32),
            pltpu.VMEM((CHUNK * HW + 16,), jnp.float32),
            pltpu.VMEM((CHUNK * HW + 16,), jnp.float32),
            pltpu.VMEM((32,), jnp.float32),
            pltpu.VMEM((32,), jnp.int32),
            pltpu.SemaphoreType.DMA,
            pltpu.SemaphoreType.DMA,
            pltpu.SemaphoreType.DMA,
            pltpu.SemaphoreType.DMA,
        ],
    )(x1, t1)


def _tc_body(x_ref, t_ref, ii_ref, o_ref):
    xb = x_ref[...]
    t = t_ref[...]
    ii = ii_ref[...]
    m = jnp.max(xb, axis=(1, 2), keepdims=True)
    idx = jnp.min(jnp.where(xb == m, ii, HW), axis=(1, 2), keepdims=True)
    mh = idx // W
    mw = idx - mh * W
    h1 = jnp.maximum(mh - HALF, 0)
    h2 = jnp.minimum(mh + HALF, H - 1)
    w1 = jnp.maximum(mw - HALF, 0)
    w2 = jnp.minimum(mw + HALF, W - 1)
    area = (h2 - h1 + 1) * (w2 - w1 + 1)
    lam = jnp.float32(HW) / (jnp.float32(HW) - area.astype(jnp.float32))
    on = t != 0.0
    lam_eff = jnp.where(on, lam, jnp.float32(1))
    a = jnp.where(on, lam, jnp.float32(0))
    hh = lax.broadcasted_iota(jnp.int32, (RB, H, 1), 1)
    ww = lax.broadcasted_iota(jnp.int32, (RB, 1, W), 2)
    rowm = ((hh >= h1) & (hh <= h2)).astype(jnp.float32) * a
    colm = ((ww >= w1) & (ww <= w2)).astype(jnp.float32)
    g = lam_eff - rowm * colm
    o_ref[...] = xb * g


def _tc_call(x2, t2):
    ii = jnp.arange(HW, dtype=jnp.int32).reshape(1, H, W)
    return pl.pallas_call(
        _tc_body,
        grid=(R // RB,),
        in_specs=[
            pl.BlockSpec((RB, H, W), lambda i: (i, 0, 0)),
            pl.BlockSpec((RB, 1, 1), lambda i: (i, 0, 0)),
            pl.BlockSpec((1, H, W), lambda i: (0, 0, 0)),
        ],
        out_specs=pl.BlockSpec((RB, H, W), lambda i: (i, 0, 0)),
        out_shape=jax.ShapeDtypeStruct((R, H, W), jnp.float32),
    )(x2, t2, ii)


def kernel(x, T):
    b, c, h, w = x.shape
    out = _tc_call(x.reshape(R, H, W), T.reshape(R, 1, 1))
    return out.reshape(b, c, h, w)
